# baseline (device time: 74672 ns/iter reference)
import jax
import jax.numpy as jnp
from jax import lax
from jax.experimental import pallas as pl
from jax.experimental.pallas import tpu as pltpu

N_DEV = 8
B, SQ, SKV = 2, 512, 512
HQ_LOC, DH = 8, 64
EMB = 768
ROWS = B * SQ
R = ROWS // N_DEV
NG = 3
COLS = EMB // NG

RS_SCHED = (
    ((0, 4, ((0, 4),)), (4, 2, ((0, 2),)), (6, 1, ((0, 1),))),
    ((0, 2, ((0, 2), (4, 2))), (2, 1, ((0, 1), (4, 1))), (3, 4, ((0, 1),))),
    ((0, 1, ((0, 1), (2, 1), (4, 1), (6, 1))), (1, 4, ((0, 1), (2, 1))),
     (5, 2, ((0, 1),))),
)
AG_SCHED = (
    ((7, 1, ((0, 1),)), (6, 2, ((0, 2),)), (4, 4, ((0, 4),))),
    ((7, 4, ((0, 1),)), (3, 1, ((0, 1), (4, 1))), (2, 2, ((0, 2), (4, 2)))),
    ((7, 2, ((0, 1),)), (5, 4, ((0, 1), (2, 1))),
     (1, 1, ((0, 1), (2, 1), (4, 1), (6, 1)))),
)
RBUF_BASE = (0, 4, 6)
N_RDMA = 15


def _allreduce_body(p_ref, out_ref, rbuf, ss_rs, rs_rs, ss_ag, rs_ag):
    my = lax.axis_index("i")
    v = my ^ ((my >> 1) & 1)

    def phys(u):
        return u ^ ((u >> 1) & 1)

    partners = {m: phys(v ^ m) for m in (1, 2, 4)}

    barrier = pltpu.get_barrier_semaphore()
    for m in (1, 2, 4):
        pl.semaphore_signal(
            barrier, inc=1,
            device_id=(partners[m],), device_id_type=pl.DeviceIdType.MESH,
        )
    pl.semaphore_wait(barrier, 3)

    out_ref[...] = p_ref[...]

    def gcols(g):
        return pl.ds(g * COLS, COLS)

    rs_idx = [0]
    ag_idx = [0]

    def issue_rs(g, j):
        fixedmask, m, runs = RS_SCHED[g][j]
        send_base = (v & fixedmask) | ((v ^ m) & m)
        descs = []
        slot = RBUF_BASE[j]
        for off, n in runs:
            i = rs_idx[0]
            rs_idx[0] += 1
            rdma = pltpu.make_async_remote_copy(
                src_ref=out_ref.at[pl.ds((send_base + off) * R, n * R), gcols(g)],
                dst_ref=rbuf.at[pl.ds(slot * R, n * R), gcols(g)],
                send_sem=ss_rs.at[i],
                recv_sem=rs_rs.at[i],
                device_id=(partners[m],),
                device_id_type=pl.DeviceIdType.MESH,
            )
            rdma.start()
            descs.append(rdma)
            slot += n
        return descs

    def add_rs(g, j):
        fixedmask, m, runs = RS_SCHED[g][j]
        keep_base = (v & fixedmask) | (v & m)
        slot = RBUF_BASE[j]
        for off, n in runs:
            rows = pl.ds((keep_base + off) * R, n * R)
            out_ref[rows, gcols(g)] = (
                out_ref[rows, gcols(g)]
                + rbuf[pl.ds(slot * R, n * R), gcols(g)]
            )
            slot += n

    def issue_ag(g, j):
        validmask, m, runs = AG_SCHED[g][j]
        base = v & validmask
        descs = []
        for off, n in runs:
            i = ag_idx[0]
            ag_idx[0] += 1
            rows = pl.ds((base + off) * R, n * R)
            rdma = pltpu.make_async_remote_copy(
                src_ref=out_ref.at[rows, gcols(g)],
                dst_ref=out_ref.at[rows, gcols(g)],
                send_sem=ss_ag.at[i],
                recv_sem=rs_ag.at[i],
                device_id=(partners[m],),
                device_id_type=pl.DeviceIdType.MESH,
            )
            rdma.start()
            descs.append(rdma)
        return descs

    pend = {g: issue_rs(g, 0) for g in range(NG)}
    ag_pend = {}
    for j in range(3):
        for g in range(NG):
            for d in pend[g]:
                d.wait()
            add_rs(g, j)
            if j < 2:
                pend[g] = issue_rs(g, j + 1)
            else:
                ag_pend[g] = issue_ag(g, 0)

    for j in range(3):
        for g in range(NG):
            for d in ag_pend[g]:
                d.wait()
            if j < 2:
                ag_pend[g] = issue_ag(g, j + 1)


def _cube_allreduce(partial2d):
    return pl.pallas_call(
        _allreduce_body,
        out_shape=jax.ShapeDtypeStruct((ROWS, EMB), jnp.bfloat16),
        in_specs=[pl.BlockSpec(memory_space=pltpu.VMEM)],
        out_specs=pl.BlockSpec(memory_space=pltpu.VMEM),
        scratch_shapes=[
            pltpu.VMEM((7 * R, EMB), jnp.bfloat16),
            pltpu.SemaphoreType.DMA((N_RDMA,)),
            pltpu.SemaphoreType.DMA((N_RDMA,)),
            pltpu.SemaphoreType.DMA((N_RDMA,)),
            pltpu.SemaphoreType.DMA((N_RDMA,)),
        ],
        compiler_params=pltpu.CompilerParams(collective_id=0),
    )(partial2d)


QBLK = 64


def _attn_body(x_ref, wq_ref, k_ref, v_ref, wo_ref, out_ref, q_scr, ctx_scr):
    q_scr[...] = jnp.dot(
        x_ref[...], wq_ref[...], preferred_element_type=jnp.float32
    ).astype(jnp.bfloat16)

    for b in range(B):
        for h in range(HQ_LOC):
            bh = b * HQ_LOC + h
            for qi in range(SQ // QBLK):
                klen = (qi + 1) * QBLK
                qblk = q_scr[pl.ds(b * SQ + qi * QBLK, QBLK), pl.ds(h * DH, DH)]
                s = lax.dot_general(
                    qblk, k_ref[bh, pl.ds(0, klen), :],
                    (((1,), (1,)), ((), ())),
                    preferred_element_type=jnp.float32,
                )
                m = jnp.max(s, axis=-1, keepdims=True)
                e = jnp.exp(s - m)
                p = (e / jnp.sum(e, axis=-1, keepdims=True)).astype(jnp.bfloat16)
                o = jnp.dot(
                    p, v_ref[bh, pl.ds(0, klen), :],
                    preferred_element_type=jnp.float32,
                )
                ctx_scr[pl.ds(b * SQ + qi * QBLK, QBLK), pl.ds(h * DH, DH)] = (
                    o.astype(jnp.bfloat16)
                )

    out_ref[...] = jnp.dot(
        ctx_scr[...], wo_ref[...], preferred_element_type=jnp.float32
    ).astype(jnp.bfloat16)


def _fused_attn(xb, wqb, kb, vb, wob):
    return pl.pallas_call(
        _attn_body,
        out_shape=jax.ShapeDtypeStruct((ROWS, EMB), jnp.bfloat16),
        in_specs=[pl.BlockSpec(memory_space=pltpu.VMEM)] * 5,
        out_specs=pl.BlockSpec(memory_space=pltpu.VMEM),
        scratch_shapes=[
            pltpu.VMEM((ROWS, HQ_LOC * DH), jnp.bfloat16),
            pltpu.VMEM((ROWS, HQ_LOC * DH), jnp.bfloat16),
        ],
    )(xb, wqb, kb, vb, wob)


def kernel(x, Wq, K_ext, V_ext, Wo):
    my = lax.axis_index("i")

    xb = x.astype(jnp.bfloat16).reshape(ROWS, EMB)
    wqb = (Wq * 0.125).astype(jnp.bfloat16)
    wob = Wo.astype(jnp.bfloat16)
    k = lax.dynamic_slice_in_dim(K_ext, my * HQ_LOC, HQ_LOC, axis=2)
    v = lax.dynamic_slice_in_dim(V_ext, my * HQ_LOC, HQ_LOC, axis=2)
    kb = k.astype(jnp.bfloat16).transpose(0, 2, 1, 3).reshape(
        B * HQ_LOC, SKV, DH
    )
    vb = v.astype(jnp.bfloat16).transpose(0, 2, 1, 3).reshape(
        B * HQ_LOC, SKV, DH
    )

    partial = _fused_attn(xb, wqb, kb, vb, wob)
    out = _cube_allreduce(partial)
    return out.reshape(B, SQ, EMB).astype(jnp.float32)


# device time: 55223 ns/iter; 1.3522x vs baseline; 1.3522x over previous
import jax
import jax.numpy as jnp
from jax import lax
from jax.experimental import pallas as pl
from jax.experimental.pallas import tpu as pltpu

N_DEV = 8
B, SQ, SKV = 2, 512, 512
HQ_LOC, DH = 8, 64
HD = HQ_LOC * DH
EMB = 768
ROWS = B * SQ
R = ROWS // N_DEV
NG = 3
COLS = EMB // NG

RS_SCHED = (
    ((0, 4, ((0, 4),)), (4, 2, ((0, 2),)), (6, 1, ((0, 1),))),
    ((0, 2, ((0, 2), (4, 2))), (2, 1, ((0, 1), (4, 1))), (3, 4, ((0, 1),))),
    ((0, 1, ((0, 1), (2, 1), (4, 1), (6, 1))), (1, 4, ((0, 1), (2, 1))),
     (5, 2, ((0, 1),))),
)
AG_SCHED = (
    ((7, 1, ((0, 1),)), (6, 2, ((0, 2),)), (4, 4, ((0, 4),))),
    ((7, 4, ((0, 1),)), (3, 1, ((0, 1), (4, 1))), (2, 2, ((0, 2), (4, 2)))),
    ((7, 2, ((0, 1),)), (5, 4, ((0, 1), (2, 1))),
     (1, 1, ((0, 1), (2, 1), (4, 1), (6, 1)))),
)
RBUF_BASE = (0, 4, 6)
N_RDMA = 15


def _mm_allreduce_body(ctx_ref, wo_ref, out_ref, rbuf, ss_rs, rs_rs, ss_ag, rs_ag):
    my = lax.axis_index("i")
    v = my ^ ((my >> 1) & 1)

    def phys(u):
        return u ^ ((u >> 1) & 1)

    partners = {m: phys(v ^ m) for m in (1, 2, 4)}

    barrier = pltpu.get_barrier_semaphore()
    for m in (1, 2, 4):
        pl.semaphore_signal(
            barrier, inc=1,
            device_id=(partners[m],), device_id_type=pl.DeviceIdType.MESH,
        )
    pl.semaphore_wait(barrier, 3)

    def gcols(g):
        return pl.ds(g * COLS, COLS)

    rs_idx = [0]
    ag_idx = [0]

    def issue_rs(g, j):
        fixedmask, m, runs = RS_SCHED[g][j]
        send_base = (v & fixedmask) | ((v ^ m) & m)
        descs = []
        slot = RBUF_BASE[j]
        for off, n in runs:
            i = rs_idx[0]
            rs_idx[0] += 1
            rdma = pltpu.make_async_remote_copy(
                src_ref=out_ref.at[pl.ds((send_base + off) * R, n * R), gcols(g)],
                dst_ref=rbuf.at[pl.ds(slot * R, n * R), gcols(g)],
                send_sem=ss_rs.at[i],
                recv_sem=rs_rs.at[i],
                device_id=(partners[m],),
                device_id_type=pl.DeviceIdType.MESH,
            )
            rdma.start()
            descs.append(rdma)
            slot += n
        return descs

    def add_rs(g, j):
        fixedmask, m, runs = RS_SCHED[g][j]
        keep_base = (v & fixedmask) | (v & m)
        slot = RBUF_BASE[j]
        for off, n in runs:
            rows = pl.ds((keep_base + off) * R, n * R)
            out_ref[rows, gcols(g)] = (
                out_ref[rows, gcols(g)]
                + rbuf[pl.ds(slot * R, n * R), gcols(g)]
            )
            slot += n

    def issue_ag(g, j):
        validmask, m, runs = AG_SCHED[g][j]
        base = v & validmask
        descs = []
        for off, n in runs:
            i = ag_idx[0]
            ag_idx[0] += 1
            rows = pl.ds((base + off) * R, n * R)
            rdma = pltpu.make_async_remote_copy(
                src_ref=out_ref.at[rows, gcols(g)],
                dst_ref=out_ref.at[rows, gcols(g)],
                send_sem=ss_ag.at[i],
                recv_sem=rs_ag.at[i],
                device_id=(partners[m],),
                device_id_type=pl.DeviceIdType.MESH,
            )
            rdma.start()
            descs.append(rdma)
        return descs

    half_a = ((v ^ 4) & 4) * R
    half_b = (v & 4) * R
    out_ref[pl.ds(half_a, 4 * R), :] = jnp.dot(
        ctx_ref[pl.ds(half_a, 4 * R), :], wo_ref[...],
        preferred_element_type=jnp.float32,
    ).astype(jnp.bfloat16)
    pend = {0: issue_rs(0, 0)}
    out_ref[pl.ds(half_b, 4 * R), :] = jnp.dot(
        ctx_ref[pl.ds(half_b, 4 * R), :], wo_ref[...],
        preferred_element_type=jnp.float32,
    ).astype(jnp.bfloat16)
    pend[1] = issue_rs(1, 0)
    pend[2] = issue_rs(2, 0)

    ag_pend = {}
    for j in range(3):
        for g in range(NG):
            for d in pend[g]:
                d.wait()
            add_rs(g, j)
            if j < 2:
                pend[g] = issue_rs(g, j + 1)
            else:
                ag_pend[g] = issue_ag(g, 0)

    for j in range(3):
        for g in range(NG):
            for d in ag_pend[g]:
                d.wait()
            if j < 2:
                ag_pend[g] = issue_ag(g, j + 1)


def _mm_allreduce(ctx2d, wob):
    return pl.pallas_call(
        _mm_allreduce_body,
        out_shape=jax.ShapeDtypeStruct((ROWS, EMB), jnp.bfloat16),
        in_specs=[pl.BlockSpec(memory_space=pltpu.VMEM)] * 2,
        out_specs=pl.BlockSpec(memory_space=pltpu.VMEM),
        scratch_shapes=[
            pltpu.VMEM((7 * R, EMB), jnp.bfloat16),
            pltpu.SemaphoreType.DMA((N_RDMA,)),
            pltpu.SemaphoreType.DMA((N_RDMA,)),
            pltpu.SemaphoreType.DMA((N_RDMA,)),
            pltpu.SemaphoreType.DMA((N_RDMA,)),
        ],
        compiler_params=pltpu.CompilerParams(collective_id=0),
    )(ctx2d, wob)


def _qproj_body(x_ref, wq_ref, out_ref):
    out_ref[...] = jnp.dot(
        x_ref[...], wq_ref[...], preferred_element_type=jnp.float32
    ).astype(jnp.bfloat16)


def _qproj(xb, wqb):
    return pl.pallas_call(
        _qproj_body,
        out_shape=jax.ShapeDtypeStruct((ROWS, HD), jnp.bfloat16),
        in_specs=[pl.BlockSpec(memory_space=pltpu.VMEM)] * 2,
        out_specs=pl.BlockSpec(memory_space=pltpu.VMEM),
    )(xb, wqb)


def kernel(x, Wq, K_ext, V_ext, Wo):
    my = lax.axis_index("i")

    xb = x.astype(jnp.bfloat16).reshape(ROWS, EMB)
    wqb = (Wq * 0.125).astype(jnp.bfloat16)
    wob = Wo.astype(jnp.bfloat16)

    q = _qproj(xb, wqb).reshape(B, SQ, HQ_LOC, DH)

    k = lax.dynamic_slice_in_dim(K_ext, my * HQ_LOC, HQ_LOC, axis=2)
    v = lax.dynamic_slice_in_dim(V_ext, my * HQ_LOC, HQ_LOC, axis=2)

    scores = jnp.einsum(
        "bihd,bjhd->bhij", q, k.astype(jnp.bfloat16),
        preferred_element_type=jnp.float32,
    )
    qb = jnp.arange(SQ)[:, None] // 64
    kb = jnp.arange(SKV)[None, :] // 64
    mask = kb <= qb
    scores = jnp.where(mask[None, None], scores, -1e9)
    w = jax.nn.softmax(scores, axis=-1).astype(jnp.bfloat16)

    ctx = jnp.einsum(
        "bhij,bjhd->bihd", w, v.astype(jnp.bfloat16),
        preferred_element_type=jnp.float32,
    ).reshape(ROWS, HD).astype(jnp.bfloat16)

    out = _mm_allreduce(ctx, wob)
    return out.reshape(B, SQ, EMB).astype(jnp.float32)


# device time: 54751 ns/iter; 1.3638x vs baseline; 1.0086x over previous
import jax
import jax.numpy as jnp
from jax import lax
from jax.experimental import pallas as pl
from jax.experimental.pallas import tpu as pltpu

N_DEV = 8
B, SQ, SKV = 2, 512, 512
HQ_LOC, DH = 8, 64
HD = HQ_LOC * DH
EMB = 768
ROWS = B * SQ
R = ROWS // N_DEV
NG = 3
COLS = EMB // NG
QT = 128

RS_SCHED = (
    ((0, 4, ((0, 4),)), (4, 2, ((0, 2),)), (6, 1, ((0, 1),))),
    ((0, 2, ((0, 2), (4, 2))), (2, 1, ((0, 1), (4, 1))), (3, 4, ((0, 1),))),
    ((0, 1, ((0, 1), (2, 1), (4, 1), (6, 1))), (1, 4, ((0, 1), (2, 1))),
     (5, 2, ((0, 1),))),
)
AG_SCHED = (
    ((7, 1, ((0, 1),)), (6, 2, ((0, 2),)), (4, 4, ((0, 4),))),
    ((7, 4, ((0, 1),)), (3, 1, ((0, 1), (4, 1))), (2, 2, ((0, 2), (4, 2)))),
    ((7, 2, ((0, 1),)), (5, 4, ((0, 1), (2, 1))),
     (1, 1, ((0, 1), (2, 1), (4, 1), (6, 1)))),
)
RBUF_BASE = (0, 4, 6)
N_RDMA = 15


def _body(x_ref, wq_ref, k_ref, v_ref, wo_ref, out_ref,
          ctx_scr, rbuf, ss_rs, rs_rs, ss_ag, rs_ag):
    my = lax.axis_index("i")
    v = my ^ ((my >> 1) & 1)

    def phys(u):
        return u ^ ((u >> 1) & 1)

    partners = {m: phys(v ^ m) for m in (1, 2, 4)}

    barrier = pltpu.get_barrier_semaphore()
    for m in (1, 2, 4):
        pl.semaphore_signal(
            barrier, inc=1,
            device_id=(partners[m],), device_id_type=pl.DeviceIdType.MESH,
        )
    pl.semaphore_wait(barrier, 3)

    qv = jnp.dot(
        x_ref[...], wq_ref[...], preferred_element_type=jnp.float32
    ).astype(jnp.bfloat16)

    for b in range(B):
        for h in range(HQ_LOC):
            bh = b * HQ_LOC + h
            q_bh = lax.slice(qv, (b * SQ, h * DH), ((b + 1) * SQ, (h + 1) * DH))
            for qi in range(SQ // QT):
                klen = (qi + 1) * QT
                q_tile = lax.slice(q_bh, (qi * QT, 0), ((qi + 1) * QT, DH))
                s = lax.dot_general(
                    q_tile, k_ref[bh, pl.ds(0, klen), :],
                    (((1,), (1,)), ((), ())),
                    preferred_element_type=jnp.float32,
                )
                rows = lax.broadcasted_iota(jnp.int32, (QT, klen), 0)
                cols = lax.broadcasted_iota(jnp.int32, (QT, klen), 1)
                s = jnp.where((rows < 64) & (cols >= klen - 64), -1e9, s)
                m_ = jnp.max(s, axis=-1, keepdims=True)
                e = jnp.exp(s - m_)
                p = (e / jnp.sum(e, axis=-1, keepdims=True)).astype(jnp.bfloat16)
                o = jnp.dot(
                    p, v_ref[bh, pl.ds(0, klen), :],
                    preferred_element_type=jnp.float32,
                )
                ctx_scr[bh, pl.ds(qi * QT, QT), :] = o.astype(jnp.bfloat16)

    def gcols(g):
        return pl.ds(g * COLS, COLS)

    rs_idx = [0]
    ag_idx = [0]

    def issue_rs(g, j):
        fixedmask, m, runs = RS_SCHED[g][j]
        send_base = (v & fixedmask) | ((v ^ m) & m)
        descs = []
        slot = RBUF_BASE[j]
        for off, n in runs:
            i = rs_idx[0]
            rs_idx[0] += 1
            rdma = pltpu.make_async_remote_copy(
                src_ref=out_ref.at[pl.ds((send_base + off) * R, n * R), gcols(g)],
                dst_ref=rbuf.at[pl.ds(slot * R, n * R), gcols(g)],
                send_sem=ss_rs.at[i],
                recv_sem=rs_rs.at[i],
                device_id=(partners[m],),
                device_id_type=pl.DeviceIdType.MESH,
            )
            rdma.start()
            descs.append(rdma)
            slot += n
        return descs

    def add_rs(g, j):
        fixedmask, m, runs = RS_SCHED[g][j]
        keep_base = (v & fixedmask) | (v & m)
        slot = RBUF_BASE[j]
        for off, n in runs:
            rows = pl.ds((keep_base + off) * R, n * R)
            out_ref[rows, gcols(g)] = (
                out_ref[rows, gcols(g)]
                + rbuf[pl.ds(slot * R, n * R), gcols(g)]
            )
            slot += n

    def issue_ag(g, j):
        validmask, m, runs = AG_SCHED[g][j]
        base = v & validmask
        descs = []
        for off, n in runs:
            i = ag_idx[0]
            ag_idx[0] += 1
            rows = pl.ds((base + off) * R, n * R)
            rdma = pltpu.make_async_remote_copy(
                src_ref=out_ref.at[rows, gcols(g)],
                dst_ref=out_ref.at[rows, gcols(g)],
                send_sem=ss_ag.at[i],
                recv_sem=rs_ag.at[i],
                device_id=(partners[m],),
                device_id_type=pl.DeviceIdType.MESH,
            )
            rdma.start()
            descs.append(rdma)
        return descs

    half_a = ((v ^ 4) & 4) * R
    half_b = (v & 4) * R

    def project(base):
        bsel = base // (4 * R)
        ctx_cat = jnp.concatenate(
            [ctx_scr[bsel * HQ_LOC + h] for h in range(HQ_LOC)], axis=1
        )
        out_ref[pl.ds(base, 4 * R), :] = jnp.dot(
            ctx_cat, wo_ref[...], preferred_element_type=jnp.float32
        ).astype(jnp.bfloat16)

    project(half_a)
    pend = {0: issue_rs(0, 0)}
    project(half_b)
    pend[1] = issue_rs(1, 0)
    pend[2] = issue_rs(2, 0)

    ag_pend = {}
    for j in range(3):
        for g in range(NG):
            for d in pend[g]:
                d.wait()
            add_rs(g, j)
            if j < 2:
                pend[g] = issue_rs(g, j + 1)
            else:
                ag_pend[g] = issue_ag(g, 0)

    for j in range(3):
        for g in range(NG):
            for d in ag_pend[g]:
                d.wait()
            if j < 2:
                ag_pend[g] = issue_ag(g, j + 1)


def kernel(x, Wq, K_ext, V_ext, Wo):
    my = lax.axis_index("i")

    xb = x.astype(jnp.bfloat16).reshape(ROWS, EMB)
    wqb = (Wq * 0.125).astype(jnp.bfloat16)
    wob = Wo.astype(jnp.bfloat16)
    k = lax.dynamic_slice_in_dim(K_ext, my * HQ_LOC, HQ_LOC, axis=2)
    v = lax.dynamic_slice_in_dim(V_ext, my * HQ_LOC, HQ_LOC, axis=2)
    k8 = k.astype(jnp.bfloat16).transpose(0, 2, 1, 3).reshape(B * HQ_LOC, SKV, DH)
    v8 = v.astype(jnp.bfloat16).transpose(0, 2, 1, 3).reshape(B * HQ_LOC, SKV, DH)

    out = pl.pallas_call(
        _body,
        out_shape=jax.ShapeDtypeStruct((ROWS, EMB), jnp.bfloat16),
        in_specs=[pl.BlockSpec(memory_space=pltpu.VMEM)] * 5,
        out_specs=pl.BlockSpec(memory_space=pltpu.VMEM),
        scratch_shapes=[
            pltpu.VMEM((B * HQ_LOC, SQ, DH), jnp.bfloat16),
            pltpu.VMEM((7 * R, EMB), jnp.bfloat16),
            pltpu.SemaphoreType.DMA((N_RDMA,)),
            pltpu.SemaphoreType.DMA((N_RDMA,)),
            pltpu.SemaphoreType.DMA((N_RDMA,)),
            pltpu.SemaphoreType.DMA((N_RDMA,)),
        ],
        compiler_params=pltpu.CompilerParams(collective_id=0),
    )(xb, wqb, k8, v8, wob)
    return out.reshape(B, SQ, EMB).astype(jnp.float32)
